# Initial kernel scaffold; baseline (speedup 1.0000x reference)
import jax
import jax.numpy as jnp
from jax import lax
from jax.experimental import pallas as pl
from jax.experimental.pallas import tpu as pltpu
from jax.experimental.pallas import tpu_sc as plsc

_N_ATOMS = 1048576
_N_SPECIES = 100
_D = 4

_NC = 2
_NS = 16
_NW = _NC * _NS

_A = _N_ATOMS // _NW
_C = 4096
_G = _A // _C
_C4 = _C * _D


def _body(species_hbm, table_hbm, out_hbm,
          table_v, idx0, idx1, rows0, rows1, si0, si1, so0, so1):
  cid = lax.axis_index("c")
  sid = lax.axis_index("s")
  wid = sid * _NC + cid
  base = wid * _A

  pltpu.sync_copy(table_hbm, table_v)

  idx_bufs = (idx0, idx1)
  rows_bufs = (rows0, rows1)
  sin = (si0, si1)
  sout = (so0, so1)

  def fetch(g, b):
    return pltpu.async_copy(
        species_hbm.at[pl.ds(base + g * _C, _C)], idx_bufs[b], sin[b])

  def flush(g, b):
    return pltpu.async_copy(
        rows_bufs[b], out_hbm.at[pl.ds(base * _D + g * _C4, _C4)], sout[b])

  iota4 = lax.iota(jnp.int32, 16) * 4

  cps_in = {0: fetch(0, 0), 1: fetch(1, 1)}
  cps_out = {}
  for g in range(_G):
    b = g & 1
    if g >= 2:
      cps_out[g - 2].wait()
    cps_in[g].wait()

    idx_b = idx_bufs[b]
    rows_b = rows_bufs[b]

    @pl.loop(0, _C // 16, unroll=8)
    def _(i):
      sp = idx_b[pl.ds(i * 16, 16)]
      flat = sp * _D
      pos = iota4 + i * 64
      for d in range(_D):
        vals = plsc.load_gather(table_v, [flat + d if d else flat])
        plsc.store_scatter(rows_b, [pos + d if d else pos], vals)

    cps_out[g] = flush(g, b)
    if g + 2 < _G:
      cps_in[g + 2] = fetch(g + 2, b)

  cps_out[_G - 2].wait()
  cps_out[_G - 1].wait()


@jax.jit
def _lookup(species_i32, table_flat):
  mesh = plsc.VectorSubcoreMesh(core_axis_name="c", subcore_axis_name="s")
  return pl.kernel(
      _body,
      out_type=jax.ShapeDtypeStruct((_N_ATOMS * _D,), jnp.float32),
      mesh=mesh,
      scratch_types=[
          pltpu.VMEM((_N_SPECIES * _D,), jnp.float32),
          pltpu.VMEM((_C,), jnp.int32),
          pltpu.VMEM((_C,), jnp.int32),
          pltpu.VMEM((_C4,), jnp.float32),
          pltpu.VMEM((_C4,), jnp.float32),
          pltpu.SemaphoreType.DMA,
          pltpu.SemaphoreType.DMA,
          pltpu.SemaphoreType.DMA,
          pltpu.SemaphoreType.DMA,
      ],
  )(species_i32, table_flat)


def kernel(species, embedding_weight):
  sp = species.astype(jnp.int32)
  tbl = embedding_weight.reshape(-1)
  out = _lookup(sp, tbl)
  return out.reshape(_N_ATOMS, _D)


# speedup vs baseline: 5.7581x; 5.7581x over previous
import jax
import jax.numpy as jnp
from jax import lax
from jax.experimental import pallas as pl
from jax.experimental.pallas import tpu as pltpu
from jax.experimental.pallas import tpu_sc as plsc

_N_ATOMS = 1048576
_N_SPECIES = 100
_D = 4

_NC = 2
_NS = 16
_NW = _NC * _NS

_A = _N_ATOMS // _NW
_C = 4096
_G = _A // _C
_C4 = _C * _D


def _body(species_hbm, table_hbm, out_hbm,
          table_v, idx0, idx1, rows0, rows1, si0, si1, so0, so1):
  cid = lax.axis_index("c")
  sid = lax.axis_index("s")
  wid = sid * _NC + cid
  base = wid * _A

  pltpu.sync_copy(table_hbm, table_v)

  idx_bufs = (idx0, idx1)
  rows_bufs = (rows0, rows1)
  sin = (si0, si1)
  sout = (so0, so1)

  def fetch(g, b):
    return pltpu.async_copy(
        species_hbm.at[pl.ds(base + g * _C, _C)], idx_bufs[b], sin[b])

  def flush(g, b):
    return pltpu.async_copy(
        rows_bufs[b], out_hbm.at[pl.ds(base * _D + g * _C4, _C4)], sout[b])

  iota4 = lax.iota(jnp.int32, 16) * 4

  cps_in = {0: fetch(0, 0), 1: fetch(1, 1)}
  cps_out = {}
  for g in range(_G):
    b = g & 1
    if g >= 2:
      cps_out[g - 2].wait()
    cps_in[g].wait()

    idx_b = idx_bufs[b]
    rows_b = rows_bufs[b]

    @pl.loop(0, _C // 16, unroll=8)
    def _(i):
      sp = idx_b[pl.ds(i * 16, 16)]
      flat = sp * _D
      pos = iota4 + i * 64
      for d in range(_D):
        vals = plsc.load_gather(table_v, [flat + d if d else flat])
        plsc.store_scatter(rows_b, [pos + d if d else pos], vals)

    cps_out[g] = flush(g, b)
    if g + 2 < _G:
      cps_in[g + 2] = fetch(g + 2, b)

  cps_out[_G - 2].wait()
  cps_out[_G - 1].wait()


@jax.jit
def _lookup(species_i32, table_flat):
  mesh = plsc.VectorSubcoreMesh(core_axis_name="c", subcore_axis_name="s")
  return pl.kernel(
      _body,
      out_type=jax.ShapeDtypeStruct((_N_ATOMS * _D,), jnp.float32),
      mesh=mesh,
      compiler_params=pltpu.CompilerParams(needs_layout_passes=False),
      scratch_types=[
          pltpu.VMEM((_N_SPECIES * _D,), jnp.float32),
          pltpu.VMEM((_C,), jnp.int32),
          pltpu.VMEM((_C,), jnp.int32),
          pltpu.VMEM((_C4,), jnp.float32),
          pltpu.VMEM((_C4,), jnp.float32),
          pltpu.SemaphoreType.DMA,
          pltpu.SemaphoreType.DMA,
          pltpu.SemaphoreType.DMA,
          pltpu.SemaphoreType.DMA,
      ],
  )(species_i32, table_flat)


def kernel(species, embedding_weight):
  sp = species.astype(jnp.int32)
  tbl = embedding_weight.reshape(-1)
  out = _lookup(sp, tbl)
  return out.reshape(_N_ATOMS, _D)


# speedup vs baseline: 114.4099x; 19.8693x over previous
import jax
import jax.numpy as jnp
from jax import lax
from jax.experimental import pallas as pl
from jax.experimental.pallas import tpu as pltpu
from jax.experimental.pallas import tpu_sc as plsc

_N_ATOMS = 1048576
_N_SPECIES = 100
_D = 4

_NC = 2
_NS = 16
_NW = _NC * _NS

_A = _N_ATOMS // _NW
_C = 8192
_G = _A // _C
_C4 = _C * _D


def _body(species_hbm, table_hbm, out_hbm,
          table_v, idx0, idx1, rows0, rows1, si0, si1, so0, so1):
  cid = lax.axis_index("c")
  sid = lax.axis_index("s")
  wid = sid * _NC + cid
  base = wid * _A

  pltpu.sync_copy(table_hbm, table_v)

  idx_bufs = (idx0, idx1)
  rows_bufs = (rows0, rows1)
  sin = (si0, si1)
  sout = (so0, so1)

  def fetch(g, b):
    return pltpu.async_copy(
        species_hbm.at[pl.ds(base + g * _C, _C)], idx_bufs[b], sin[b])

  def flush(g, b):
    return pltpu.async_copy(
        rows_bufs[b], out_hbm.at[pl.ds(base * _D + g * _C4, _C4)], sout[b])

  cps_in = {0: fetch(0, 0), 1: fetch(1, 1)}
  cps_out = {}
  for g in range(_G):
    b = g & 1
    if g >= 2:
      cps_out[g - 2].wait()
    cps_in[g].wait()

    idx_b = idx_bufs[b]
    rows_b = rows_bufs[b]

    @plsc.parallel_loop(0, _C // 128, 1, unroll=2)
    def _(m):
      for j in range(8):
        sp = idx_b[pl.ds(m * 128 + j * 16, 16)]
        flat = sp * _D
        for d in range(_D):
          vals = plsc.load_gather(table_v, [flat + d if d else flat])
          rows_b[pl.ds(m * 512 + d * 128 + j * 16, 16)] = vals

    cps_out[g] = flush(g, b)
    if g + 2 < _G:
      cps_in[g + 2] = fetch(g + 2, b)

  cps_out[_G - 2].wait()
  cps_out[_G - 1].wait()


@jax.jit
def _lookup(species_i32, table_flat):
  mesh = plsc.VectorSubcoreMesh(core_axis_name="c", subcore_axis_name="s")
  return pl.kernel(
      _body,
      out_type=jax.ShapeDtypeStruct((_N_ATOMS * _D,), jnp.float32),
      mesh=mesh,
      compiler_params=pltpu.CompilerParams(needs_layout_passes=False),
      scratch_types=[
          pltpu.VMEM((_N_SPECIES * _D,), jnp.float32),
          pltpu.VMEM((_C,), jnp.int32),
          pltpu.VMEM((_C,), jnp.int32),
          pltpu.VMEM((_C4,), jnp.float32),
          pltpu.VMEM((_C4,), jnp.float32),
          pltpu.SemaphoreType.DMA,
          pltpu.SemaphoreType.DMA,
          pltpu.SemaphoreType.DMA,
          pltpu.SemaphoreType.DMA,
      ],
  )(species_i32, table_flat)


def kernel(species, embedding_weight):
  sp = species.astype(jnp.int32)
  tbl = embedding_weight.reshape(-1)
  out = _lookup(sp, tbl)
  out = out.reshape(_N_ATOMS // 128, _D, 128)
  return jnp.swapaxes(out, 1, 2).reshape(_N_ATOMS, _D)
